# baseline (speedup 1.0000x reference)
import functools

import jax
import jax.numpy as jnp
from jax import lax
from jax.experimental import pallas as pl
from jax.experimental.pallas import tpu as pltpu
from jax.experimental.pallas import tpu_sc as plsc

N_NODES = 10000
DIM = 32
NGRAPH = 64
NCLS = 10
N_EDGES = 320000

_NC = 2
_NS = 16
_NW = _NC * _NS
_EPW = N_EDGES // _NW
_CH = 1000
_NCHUNK = _EPW // _CH
_NPAD = 10240
_ZROWS = _NPAD // _NS


def _proj_body(x_ref, w_ref, o_ref):
    o_ref[...] = jnp.dot(x_ref[...], w_ref[...],
                         preferred_element_type=jnp.float32)


def _proj(x, w):
    return pl.pallas_call(
        _proj_body,
        out_shape=jax.ShapeDtypeStruct((x.shape[0], w.shape[1]), jnp.float32),
    )(x, w)


def _layer_body(first, inp_ref, agg_ref, w1_ref, b1_ref, w2_ref, b2_ref,
                g_ref, bt_ref, o_ref):
    u = inp_ref[...] + agg_ref[0, :N_NODES] + agg_ref[1, :N_NODES]
    if first:
        h2 = jnp.maximum(u + b1_ref[...], 0.0)
    else:
        h2 = jnp.maximum(
            jnp.dot(u, w1_ref[...], preferred_element_type=jnp.float32)
            + b1_ref[...], 0.0)
    h2 = jnp.dot(h2, w2_ref[...], preferred_element_type=jnp.float32) \
        + b2_ref[...]
    h2 = jnp.maximum(h2, 0.0)
    mu = jnp.mean(h2, axis=0, keepdims=True)
    var = jnp.mean((h2 - mu) ** 2, axis=0, keepdims=True)
    o_ref[...] = g_ref[...] * (h2 - mu) / jnp.sqrt(var + 1e-5) + bt_ref[...]


def _layer(first, inp, agg2, w1, b1, w2, b2, g, bt):
    return pl.pallas_call(
        functools.partial(_layer_body, first),
        out_shape=jax.ShapeDtypeStruct((N_NODES, DIM), jnp.float32),
    )(inp, agg2, w1, b1.reshape(1, DIM), w2, b2.reshape(1, DIM),
      g.reshape(1, DIM), bt.reshape(1, DIM))


def _head_body(h_ref, batch_ref, fc1w_ref, fc1b_ref, fc2w_ref, fc2b_ref,
               o_ref):
    h = h_ref[...]
    batch = batch_ref[...]
    gids = jax.lax.broadcasted_iota(jnp.int32, (NGRAPH, N_NODES), 0)
    onehot = jnp.where(gids == batch, 1.0, 0.0).astype(jnp.float32)
    pooled_sum = jnp.dot(onehot, h, preferred_element_type=jnp.float32)
    cnt = jnp.sum(onehot, axis=1, keepdims=True)
    pooled = pooled_sum / jnp.maximum(cnt, 1.0)
    z = jnp.maximum(
        jnp.dot(pooled, fc1w_ref[...], preferred_element_type=jnp.float32)
        + fc1b_ref[...], 0.0)
    z = jnp.dot(z, fc2w_ref[...], preferred_element_type=jnp.float32) \
        + fc2b_ref[...]
    m = jnp.max(z, axis=-1, keepdims=True)
    lse = jnp.log(jnp.sum(jnp.exp(z - m), axis=-1, keepdims=True)) + m
    o_ref[...] = z - lse


def _head(h, batch, fc1w, fc1b, fc2w, fc2b):
    return pl.pallas_call(
        _head_body,
        out_shape=jax.ShapeDtypeStruct((NGRAPH, NCLS), jnp.float32),
    )(h, batch.reshape(1, N_NODES), fc1w, fc1b.reshape(1, DIM), fc2w,
      fc2b.reshape(1, NCLS))


def _segsum_body(h_hbm, src_hbm, dst_hbm, out_hbm,
                 src_v, dst_v, rows_v, zero_v, h_sh, acc_sh,
                 gsem, ssem):
    c = lax.axis_index("c")
    s = lax.axis_index("s")
    wid = s * _NC + c

    def _load_idx(j):
        base = wid * _EPW + j * _CH
        b = j % 2
        pltpu.sync_copy(src_hbm.at[pl.ds(base, _CH)], src_v.at[b])
        pltpu.sync_copy(dst_hbm.at[pl.ds(base, _CH)], dst_v.at[b])

    _load_idx(0)
    _LAST = N_NODES - (_NS - 1) * _ZROWS

    @pl.when(s < _NS - 1)
    def _stage_full():
        pltpu.sync_copy(h_hbm.at[pl.ds(s * _ZROWS, _ZROWS)],
                        h_sh.at[pl.ds(s * _ZROWS, _ZROWS)])

    @pl.when(s == _NS - 1)
    def _stage_tail():
        pltpu.sync_copy(h_hbm.at[pl.ds((_NS - 1) * _ZROWS, _LAST)],
                        h_sh.at[pl.ds((_NS - 1) * _ZROWS, _LAST)])

    z16 = jnp.zeros((16,), jnp.float32)

    def _zero_row(i, carry):
        zero_v[i, pl.ds(0, 16)] = z16
        zero_v[i, pl.ds(16, 16)] = z16
        return carry

    lax.fori_loop(0, _ZROWS, _zero_row, 0)
    pltpu.sync_copy(zero_v, acc_sh.at[pl.ds(s * _ZROWS, _ZROWS)])
    plsc.subcore_barrier()

    gathers = [pltpu.async_copy(h_sh.at[src_v.at[0]], rows_v.at[0], gsem)]
    scatters = []
    for j in range(_NCHUNK):
        b = j % 2
        if j + 1 < _NCHUNK:
            if j - 1 >= 0:
                scatters[j - 1].wait()
            _load_idx(j + 1)
            gathers.append(pltpu.async_copy(
                h_sh.at[src_v.at[1 - b]], rows_v.at[1 - b], gsem))
        gathers[j].wait()
        scatters.append(pltpu.async_copy(
            rows_v.at[b], acc_sh.at[dst_v.at[b]], ssem, add=True))
    scatters[_NCHUNK - 2].wait()
    scatters[_NCHUNK - 1].wait()

    plsc.subcore_barrier()
    pltpu.sync_copy(acc_sh.at[pl.ds(s * _ZROWS, _ZROWS)],
                    out_hbm.at[c, pl.ds(s * _ZROWS, _ZROWS)])


_segsum_call = pl.kernel(
    _segsum_body,
    out_type=jax.ShapeDtypeStruct((_NC, _NPAD, DIM), jnp.float32),
    mesh=plsc.VectorSubcoreMesh(core_axis_name="c", subcore_axis_name="s"),
    compiler_params=pltpu.CompilerParams(use_tc_tiling_on_sc=False),
    scratch_types=[
        pltpu.VMEM((2, _CH), jnp.int32),
        pltpu.VMEM((2, _CH), jnp.int32),
        pltpu.VMEM((2, _CH, DIM), jnp.float32),
        pltpu.VMEM((_ZROWS, DIM), jnp.float32),
        pltpu.VMEM_SHARED((_NPAD, DIM), jnp.float32),
        pltpu.VMEM_SHARED((_NPAD, DIM), jnp.float32),
        pltpu.SemaphoreType.DMA,
        pltpu.SemaphoreType.DMA,
    ],
)


def _segsum(h, src, dst):
    return _segsum_call(h, src, dst)


def kernel(x, edge_index, batch,
           W1_1, b1_1, W2_1, b2_1, g_1, bt_1,
           W1_2, b1_2, W2_2, b2_2, g_2, bt_2,
           W1_3, b1_3, W2_3, b2_3, g_3, bt_3,
           W1_4, b1_4, W2_4, b2_4, g_4, bt_4,
           W1_5, b1_5, W2_5, b2_5, g_5, bt_5,
           fc1_W, fc1_b, fc2_W, fc2_b):
    src = edge_index[0]
    dst = edge_index[1]
    params = [
        (W1_1, b1_1, W2_1, b2_1, g_1, bt_1),
        (W1_2, b1_2, W2_2, b2_2, g_2, bt_2),
        (W1_3, b1_3, W2_3, b2_3, g_3, bt_3),
        (W1_4, b1_4, W2_4, b2_4, g_4, bt_4),
        (W1_5, b1_5, W2_5, b2_5, g_5, bt_5),
    ]
    h = _proj(x, W1_1)
    for l, (w1, b1, w2, b2, g, bt) in enumerate(params):
        agg2 = _segsum(h, src, dst)
        h = _layer(l == 0, h, agg2, w1, b1, w2, b2, g, bt)
    return _head(h, batch, fc1_W, fc1_b, fc2_W, fc2_b)


# speedup vs baseline: 1.5487x; 1.5487x over previous
import functools

import jax
import jax.numpy as jnp
from jax import lax
from jax.experimental import pallas as pl
from jax.experimental.pallas import tpu as pltpu
from jax.experimental.pallas import tpu_sc as plsc

N_NODES = 10000
DIM = 32
NGRAPH = 64
NCLS = 10
N_EDGES = 320000

_NC = 2
_NS = 16
_NW = _NC * _NS
_EPW = N_EDGES // _NW
_CH = 1000
_NCHUNK = _EPW // _CH
_NPAD = 10240
_ZROWS = _NPAD // _NS


_PR = N_NODES // 4
_PRP = _NPAD // 4


def _pack4(v):
    return jnp.tile(v.reshape(1, DIM), (1, 4))


def _groupsum(s):
    return s[:, 0:32] + s[:, 32:64] + s[:, 64:96] + s[:, 96:128]


def _proj_body(x_ref, w_ref, o_ref):
    o_ref[...] = jnp.dot(x_ref[...], w_ref[...],
                         preferred_element_type=jnp.float32)


def _proj(x4, w_stack):
    return pl.pallas_call(
        _proj_body,
        out_shape=jax.ShapeDtypeStruct((_PR, 128), jnp.float32),
    )(x4, w_stack)


def _layer_body(first, inp_ref, agg_ref, w1_ref, b1_ref, w2_ref, b2_ref,
                g_ref, bt_ref, o_ref):
    u = inp_ref[...] + agg_ref[0, :_PR] + agg_ref[1, :_PR]
    if first:
        h2 = jnp.maximum(u + b1_ref[...], 0.0)
    else:
        h2 = jnp.maximum(
            jnp.dot(u, w1_ref[...], preferred_element_type=jnp.float32)
            + b1_ref[...], 0.0)
    h2 = jnp.dot(h2, w2_ref[...], preferred_element_type=jnp.float32) \
        + b2_ref[...]
    h2 = jnp.maximum(h2, 0.0)
    s = jnp.sum(h2, axis=0, keepdims=True)
    mu = jnp.tile(_groupsum(s) * (1.0 / N_NODES), (1, 4))
    sq = jnp.sum((h2 - mu) ** 2, axis=0, keepdims=True)
    var = jnp.tile(_groupsum(sq) * (1.0 / N_NODES), (1, 4))
    o_ref[...] = g_ref[...] * (h2 - mu) / jnp.sqrt(var + 1e-5) + bt_ref[...]


def _layer(first, inp, agg2, wb1, b1, wb2, b2, g, bt):
    return pl.pallas_call(
        functools.partial(_layer_body, first),
        out_shape=jax.ShapeDtypeStruct((_PR, 128), jnp.float32),
    )(inp, agg2, wb1, _pack4(b1), wb2, _pack4(b2), _pack4(g), _pack4(bt))


def _head_body(h_ref, b0_ref, b1_ref, b2_ref, b3_ref,
               fc1w_ref, fc1b_ref, fc2w_ref, fc2b_ref, o_ref):
    h = h_ref[...]
    gids = jax.lax.broadcasted_iota(jnp.int32, (NGRAPH, _PR), 0)
    pooled_sum = jnp.zeros((NGRAPH, DIM), jnp.float32)
    cnt = jnp.zeros((NGRAPH, 1), jnp.float32)
    for q, b_ref in enumerate((b0_ref, b1_ref, b2_ref, b3_ref)):
        onehot = jnp.where(gids == b_ref[...], 1.0, 0.0).astype(jnp.float32)
        pooled_sum = pooled_sum + jnp.dot(
            onehot, h[:, 32 * q:32 * q + 32],
            preferred_element_type=jnp.float32)
        cnt = cnt + jnp.sum(onehot, axis=1, keepdims=True)
    pooled = pooled_sum / jnp.maximum(cnt, 1.0)
    z = jnp.maximum(
        jnp.dot(pooled, fc1w_ref[...], preferred_element_type=jnp.float32)
        + fc1b_ref[...], 0.0)
    z = jnp.dot(z, fc2w_ref[...], preferred_element_type=jnp.float32) \
        + fc2b_ref[...]
    m = jnp.max(z, axis=-1, keepdims=True)
    lse = jnp.log(jnp.sum(jnp.exp(z - m), axis=-1, keepdims=True)) + m
    o_ref[...] = z - lse


def _head(h, batch, fc1w, fc1b, fc2w, fc2b):
    bq = [batch[q::4].reshape(1, _PR) for q in range(4)]
    return pl.pallas_call(
        _head_body,
        out_shape=jax.ShapeDtypeStruct((NGRAPH, NCLS), jnp.float32),
    )(h, *bq, fc1w, fc1b.reshape(1, DIM), fc2w, fc2b.reshape(1, NCLS))


def _segsum_body(h_hbm, src_hbm, dst_hbm, out_hbm,
                 src_v, dst_v, rows_v, zero_v, acc_sh,
                 gsem, ssem):
    c = lax.axis_index("c")
    s = lax.axis_index("s")
    wid = s * _NC + c

    def _load_idx(j):
        base = wid * _EPW + j * _CH
        b = j % 2
        pltpu.sync_copy(src_hbm.at[pl.ds(base, _CH)], src_v.at[b])
        pltpu.sync_copy(dst_hbm.at[pl.ds(base, _CH)], dst_v.at[b])

    _load_idx(0)
    gather0 = pltpu.async_copy(h_hbm.at[src_v.at[0]], rows_v.at[0], gsem)

    z16 = jnp.zeros((16,), jnp.float32)

    def _zero_row(i, carry):
        zero_v[i, pl.ds(0, 16)] = z16
        zero_v[i, pl.ds(16, 16)] = z16
        return carry

    lax.fori_loop(0, _ZROWS, _zero_row, 0)
    pltpu.sync_copy(zero_v, acc_sh.at[pl.ds(s * _ZROWS, _ZROWS)])
    plsc.subcore_barrier()

    gathers = [gather0]
    scatters = []
    for j in range(_NCHUNK):
        b = j % 2
        if j + 1 < _NCHUNK:
            if j - 1 >= 0:
                scatters[j - 1].wait()
            _load_idx(j + 1)
            gathers.append(pltpu.async_copy(
                h_hbm.at[src_v.at[1 - b]], rows_v.at[1 - b], gsem))
        gathers[j].wait()
        scatters.append(pltpu.async_copy(
            rows_v.at[b], acc_sh.at[dst_v.at[b]], ssem, add=True))
    scatters[_NCHUNK - 2].wait()
    scatters[_NCHUNK - 1].wait()

    plsc.subcore_barrier()
    pltpu.sync_copy(acc_sh.at[pl.ds(s * _ZROWS, _ZROWS)],
                    out_hbm.at[c, pl.ds(s * _ZROWS, _ZROWS)])


@functools.cache
def _make_segsum_call():
    return pl.kernel(
        _segsum_body,
        out_type=jax.ShapeDtypeStruct((_NC, _NPAD, DIM), jnp.float32),
        mesh=plsc.VectorSubcoreMesh(core_axis_name="c",
                                    subcore_axis_name="s"),
        compiler_params=pltpu.CompilerParams(use_tc_tiling_on_sc=False),
        scratch_types=[
            pltpu.VMEM((2, _CH), jnp.int32),
            pltpu.VMEM((2, _CH), jnp.int32),
            pltpu.VMEM((2, _CH, DIM), jnp.float32),
            pltpu.VMEM((_ZROWS, DIM), jnp.float32),
            pltpu.VMEM_SHARED((_NPAD, DIM), jnp.float32),
            pltpu.SemaphoreType.DMA,
            pltpu.SemaphoreType.DMA,
        ],
    )


def _segsum(h, src, dst):
    return _make_segsum_call()(h, src, dst)


def kernel(x, edge_index, batch,
           W1_1, b1_1, W2_1, b2_1, g_1, bt_1,
           W1_2, b1_2, W2_2, b2_2, g_2, bt_2,
           W1_3, b1_3, W2_3, b2_3, g_3, bt_3,
           W1_4, b1_4, W2_4, b2_4, g_4, bt_4,
           W1_5, b1_5, W2_5, b2_5, g_5, bt_5,
           fc1_W, fc1_b, fc2_W, fc2_b):
    src = edge_index[0]
    dst = edge_index[1]
    eye4 = jnp.eye(4, dtype=jnp.float32)
    params = [
        (W1_1, b1_1, W2_1, b2_1, g_1, bt_1),
        (W1_2, b1_2, W2_2, b2_2, g_2, bt_2),
        (W1_3, b1_3, W2_3, b2_3, g_3, bt_3),
        (W1_4, b1_4, W2_4, b2_4, g_4, bt_4),
        (W1_5, b1_5, W2_5, b2_5, g_5, bt_5),
    ]
    x4 = x.reshape(_PR, 4 * 128)
    h = _proj(x4, jnp.kron(eye4, W1_1))
    for l, (w1, b1, w2, b2, g, bt) in enumerate(params):
        agg2 = _segsum(h.reshape(N_NODES, DIM), src, dst)
        agg2p = agg2.reshape(_NC, _PRP, 128)
        h = _layer(l == 0, h, agg2p, jnp.kron(eye4, w1), b1,
                   jnp.kron(eye4, w2), b2, g, bt)
    return _head(h, batch, fc1_W, fc1_b, fc2_W, fc2_b)


# speedup vs baseline: 1.5753x; 1.0172x over previous
import functools

import jax
import jax.numpy as jnp
from jax import lax
from jax.experimental import pallas as pl
from jax.experimental.pallas import tpu as pltpu
from jax.experimental.pallas import tpu_sc as plsc

N_NODES = 10000
DIM = 32
NGRAPH = 64
NCLS = 10
N_EDGES = 320000

_NC = 2
_NS = 16
_NW = _NC * _NS
_EPW = N_EDGES // _NW
_CH = 1000
_NCHUNK = _EPW // _CH
_NPAD = 10240
_ZROWS = _NPAD // _NS


_PR = N_NODES // 4
_PRP = _NPAD // 4


def _pack4(v):
    return jnp.tile(v.reshape(1, DIM), (1, 4))


def _groupsum(s):
    return s[:, 0:32] + s[:, 32:64] + s[:, 64:96] + s[:, 96:128]


def _proj_body(x_ref, w_ref, o_ref):
    o_ref[...] = jnp.dot(x_ref[...], w_ref[...],
                         preferred_element_type=jnp.float32)


def _proj(x4, w_stack):
    return pl.pallas_call(
        _proj_body,
        out_shape=jax.ShapeDtypeStruct((_PR, 128), jnp.float32),
    )(x4, w_stack)


def _edges_body(ei_ref, src_ref, dst_ref):
    src_ref[...] = ei_ref[0, :]
    dst_ref[...] = ei_ref[1, :]


def _edges(edge_index):
    return pl.pallas_call(
        _edges_body,
        out_shape=[jax.ShapeDtypeStruct((N_EDGES,), jnp.int32),
                   jax.ShapeDtypeStruct((N_EDGES,), jnp.int32)],
    )(edge_index)


def _layer_body(first, inp_ref, agg_ref, w1_ref, b1_ref, w2_ref, b2_ref,
                g_ref, bt_ref, o_ref):
    u = inp_ref[...] + agg_ref[0, :_PR] + agg_ref[1, :_PR]
    if first:
        h2 = jnp.maximum(u + b1_ref[...], 0.0)
    else:
        h2 = jnp.maximum(
            jnp.dot(u, w1_ref[...], preferred_element_type=jnp.float32)
            + b1_ref[...], 0.0)
    h2 = jnp.dot(h2, w2_ref[...], preferred_element_type=jnp.float32) \
        + b2_ref[...]
    h2 = jnp.maximum(h2, 0.0)
    s = jnp.sum(h2, axis=0, keepdims=True)
    mu = jnp.tile(_groupsum(s) * (1.0 / N_NODES), (1, 4))
    sq = jnp.sum((h2 - mu) ** 2, axis=0, keepdims=True)
    var = jnp.tile(_groupsum(sq) * (1.0 / N_NODES), (1, 4))
    o_ref[...] = g_ref[...] * (h2 - mu) / jnp.sqrt(var + 1e-5) + bt_ref[...]


def _layer(first, inp, agg2, wb1, b1, wb2, b2, g, bt):
    return pl.pallas_call(
        functools.partial(_layer_body, first),
        out_shape=jax.ShapeDtypeStruct((_PR, 128), jnp.float32),
    )(inp, agg2, wb1, _pack4(b1), wb2, _pack4(b2), _pack4(g), _pack4(bt))


def _head_body(h_ref, b0_ref, b1_ref, b2_ref, b3_ref,
               fc1w_ref, fc1b_ref, fc2w_ref, fc2b_ref, o_ref):
    h = h_ref[...]
    gids = jax.lax.broadcasted_iota(jnp.int32, (NGRAPH, _PR), 0)
    pooled_sum = jnp.zeros((NGRAPH, DIM), jnp.float32)
    cnt = jnp.zeros((NGRAPH, 1), jnp.float32)
    for q, b_ref in enumerate((b0_ref, b1_ref, b2_ref, b3_ref)):
        onehot = jnp.where(gids == b_ref[...], 1.0, 0.0).astype(jnp.float32)
        pooled_sum = pooled_sum + jnp.dot(
            onehot, h[:, 32 * q:32 * q + 32],
            preferred_element_type=jnp.float32)
        cnt = cnt + jnp.sum(onehot, axis=1, keepdims=True)
    pooled = pooled_sum / jnp.maximum(cnt, 1.0)
    z = jnp.maximum(
        jnp.dot(pooled, fc1w_ref[...], preferred_element_type=jnp.float32)
        + fc1b_ref[...], 0.0)
    z = jnp.dot(z, fc2w_ref[...], preferred_element_type=jnp.float32) \
        + fc2b_ref[...]
    m = jnp.max(z, axis=-1, keepdims=True)
    lse = jnp.log(jnp.sum(jnp.exp(z - m), axis=-1, keepdims=True)) + m
    o_ref[...] = z - lse


def _head(h, batch, fc1w, fc1b, fc2w, fc2b):
    bq = [batch[q::4].reshape(1, _PR) for q in range(4)]
    return pl.pallas_call(
        _head_body,
        out_shape=jax.ShapeDtypeStruct((NGRAPH, NCLS), jnp.float32),
    )(h, *bq, fc1w, fc1b.reshape(1, DIM), fc2w, fc2b.reshape(1, NCLS))


def _segsum_body(h_hbm, src_hbm, dst_hbm, zeros_hbm, out_hbm,
                 src_v, dst_v, rows_v, acc_sh,
                 gsem, ssem):
    c = lax.axis_index("c")
    s = lax.axis_index("s")
    wid = s * _NC + c

    def _load_idx(j):
        base = wid * _EPW + j * _CH
        b = j % 2
        pltpu.sync_copy(src_hbm.at[pl.ds(base, _CH)], src_v.at[b])
        pltpu.sync_copy(dst_hbm.at[pl.ds(base, _CH)], dst_v.at[b])

    _load_idx(0)
    gather0 = pltpu.async_copy(h_hbm.at[src_v.at[0]], rows_v.at[0], gsem)

    pltpu.sync_copy(zeros_hbm, acc_sh.at[pl.ds(s * _ZROWS, _ZROWS)])
    plsc.subcore_barrier()

    gathers = [gather0]
    scatters = []
    for j in range(_NCHUNK):
        b = j % 2
        if j + 1 < _NCHUNK:
            if j - 1 >= 0:
                scatters[j - 1].wait()
            _load_idx(j + 1)
            gathers.append(pltpu.async_copy(
                h_hbm.at[src_v.at[1 - b]], rows_v.at[1 - b], gsem))
        gathers[j].wait()
        scatters.append(pltpu.async_copy(
            rows_v.at[b], acc_sh.at[dst_v.at[b]], ssem, add=True))
    scatters[_NCHUNK - 2].wait()
    scatters[_NCHUNK - 1].wait()

    plsc.subcore_barrier()
    pltpu.sync_copy(acc_sh.at[pl.ds(s * _ZROWS, _ZROWS)],
                    out_hbm.at[c, pl.ds(s * _ZROWS, _ZROWS)])


@functools.cache
def _make_segsum_call():
    return pl.kernel(
        _segsum_body,
        out_type=jax.ShapeDtypeStruct((_NC, _NPAD, DIM), jnp.float32),
        mesh=plsc.VectorSubcoreMesh(core_axis_name="c",
                                    subcore_axis_name="s"),
        compiler_params=pltpu.CompilerParams(use_tc_tiling_on_sc=False),
        scratch_types=[
            pltpu.VMEM((2, _CH), jnp.int32),
            pltpu.VMEM((2, _CH), jnp.int32),
            pltpu.VMEM((2, _CH, DIM), jnp.float32),
            pltpu.VMEM_SHARED((_NPAD, DIM), jnp.float32),
            pltpu.SemaphoreType.DMA,
            pltpu.SemaphoreType.DMA,
        ],
    )


def _segsum(h, src, dst, zeros):
    return _make_segsum_call()(h, src, dst, zeros)


def kernel(x, edge_index, batch,
           W1_1, b1_1, W2_1, b2_1, g_1, bt_1,
           W1_2, b1_2, W2_2, b2_2, g_2, bt_2,
           W1_3, b1_3, W2_3, b2_3, g_3, bt_3,
           W1_4, b1_4, W2_4, b2_4, g_4, bt_4,
           W1_5, b1_5, W2_5, b2_5, g_5, bt_5,
           fc1_W, fc1_b, fc2_W, fc2_b):
    src, dst = _edges(edge_index)
    zeros = jnp.zeros((_ZROWS, DIM), jnp.float32)
    eye4 = jnp.eye(4, dtype=jnp.float32)
    params = [
        (W1_1, b1_1, W2_1, b2_1, g_1, bt_1),
        (W1_2, b1_2, W2_2, b2_2, g_2, bt_2),
        (W1_3, b1_3, W2_3, b2_3, g_3, bt_3),
        (W1_4, b1_4, W2_4, b2_4, g_4, bt_4),
        (W1_5, b1_5, W2_5, b2_5, g_5, bt_5),
    ]
    x4 = x.reshape(_PR, 4 * 128)
    h = _proj(x4, jnp.kron(eye4, W1_1))
    for l, (w1, b1, w2, b2, g, bt) in enumerate(params):
        agg2 = _segsum(h.reshape(N_NODES, DIM), src, dst, zeros)
        agg2p = agg2.reshape(_NC, _PRP, 128)
        h = _layer(l == 0, h, agg2p, jnp.kron(eye4, w1), b1,
                   jnp.kron(eye4, w2), b2, g, bt)
    return _head(h, batch, fc1_W, fc1_b, fc2_W, fc2_b)
